# baseline (speedup 1.0000x reference)
import functools

import jax
import jax.numpy as jnp
from jax import lax
from jax.experimental import pallas as pl
from jax.experimental.pallas import tpu as pltpu
from jax.experimental.pallas import tpu_sc as plsc

_B, _N, _J = 16, 20000, 17
_S = 4800
_DP = _J * 3
_L = 16
_NC, _NS = 2, 16
_NW = _NC * _NS
_BLK = 80
_NBLK = _S // _BLK
_ROUNDS = -(-_NBLK // _NW)
_G = _BLK // _L


def _make_gather():
    mesh = plsc.VectorSubcoreMesh(core_axis_name="c", subcore_axis_name="s")

    @functools.partial(
        pl.kernel,
        mesh=mesh,
        compiler_params=pltpu.CompilerParams(use_tc_tiling_on_sc=False),
        out_type=(
            jax.ShapeDtypeStruct((_S,), jnp.float32),
            jax.ShapeDtypeStruct((_NBLK, 4, _BLK), jnp.float32),
            jax.ShapeDtypeStruct((_S,), jnp.float32),
            jax.ShapeDtypeStruct((_NBLK, _DP, _BLK), jnp.float32),
        ),
        scratch_types=[
            pltpu.VMEM((_BLK,), jnp.int32),
            pltpu.VMEM((_BLK,), jnp.int32),
            pltpu.VMEM((_BLK,), jnp.int32),
            pltpu.VMEM((_BLK,), jnp.int32),
            pltpu.VMEM((4, _BLK), jnp.int32),
            pltpu.VMEM((_DP, _BLK), jnp.int32),
            pltpu.VMEM((_BLK,), jnp.float32),
            pltpu.VMEM((4, _BLK), jnp.float32),
            pltpu.VMEM((_BLK,), jnp.float32),
            pltpu.VMEM((_DP, _BLK), jnp.float32),
            pltpu.SemaphoreType.DMA,
        ],
    )
    def gather_kernel(boxes_hbm, scores_hbm, joints_hbm,
                      selb_hbm, sell_hbm, selx_hbm,
                      out_b, out_boxes, out_sc, out_pose,
                      b_v, l_v, x_v, sidx_v, bidx_v, pidx_v,
                      bf_v, boxg_v, scg_v, poseg_v, sem):
        wid = lax.axis_index("s") * _NC + lax.axis_index("c")
        for t in range(_ROUNDS):
            blk = wid + _NW * t

            @pl.when(blk < _NBLK)
            def _round():
                base = blk * _BLK
                pltpu.sync_copy(selb_hbm.at[pl.ds(base, _BLK)], b_v)
                pltpu.sync_copy(sell_hbm.at[pl.ds(base, _BLK)], l_v)
                pltpu.sync_copy(selx_hbm.at[pl.ds(base, _BLK)], x_v)
                for i in range(_G):
                    sl = pl.ds(i * _L, _L)
                    b = b_v[sl]
                    flat = b * _N + x_v[sl]
                    bf_v[sl] = b.astype(jnp.float32)
                    sidx_v[sl] = flat + l_v[sl]
                    f4 = flat * 4
                    for c in range(4):
                        bidx_v[c, sl] = f4 + c
                    f51 = flat * _DP
                    for c in range(_DP):
                        pidx_v[c, sl] = f51 + c
                pend = [pltpu.async_copy(scores_hbm.at[sidx_v], scg_v, sem)]
                for c in range(4):
                    pend.append(pltpu.async_copy(
                        boxes_hbm.at[bidx_v.at[c]], boxg_v.at[c], sem))
                for c in range(_DP):
                    pend.append(pltpu.async_copy(
                        joints_hbm.at[pidx_v.at[c]], poseg_v.at[c], sem))
                for d in pend:
                    d.wait()
                pltpu.sync_copy(bf_v, out_b.at[pl.ds(base, _BLK)])
                pltpu.sync_copy(boxg_v, out_boxes.at[blk])
                pltpu.sync_copy(scg_v, out_sc.at[pl.ds(base, _BLK)])
                pltpu.sync_copy(poseg_v, out_pose.at[blk])

    return gather_kernel


_gather = _make_gather()


@jax.jit
def kernel(pred_boxes, pred_scores, pred_joints, selected_indexes):
    boxes_flat = pred_boxes.reshape(_B * _N * 4)
    scores_flat = pred_scores.reshape(_B * _N)
    joints_flat = pred_joints.reshape(_B * _N * _DP)
    bf, boxes_t, sc, pose_t = _gather(
        boxes_flat, scores_flat, joints_flat,
        selected_indexes[:, 0], selected_indexes[:, 1], selected_indexes[:, 2])
    boxes = boxes_t.transpose(0, 2, 1).reshape(_S, 4)
    pose = pose_t.transpose(0, 2, 1).reshape(_S, _DP)
    return jnp.concatenate([bf[:, None], boxes, sc[:, None], pose], axis=1)


# speedup vs baseline: 116.6753x; 116.6753x over previous
import functools

import jax
import jax.numpy as jnp
from jax import lax
from jax.experimental import pallas as pl
from jax.experimental.pallas import tpu as pltpu
from jax.experimental.pallas import tpu_sc as plsc

_B, _N, _J = 16, 20000, 17
_S = 4800
_DP = _J * 3
_L = 16
_NC, _NS = 2, 16
_NW = _NC * _NS
_BLK = 80
_NBLK = _S // _BLK
_ROUNDS = -(-_NBLK // _NW)
_G = _BLK // _L


def _make_gather():
    mesh = plsc.VectorSubcoreMesh(core_axis_name="c", subcore_axis_name="s")

    @functools.partial(
        pl.kernel,
        mesh=mesh,
        compiler_params=pltpu.CompilerParams(use_tc_tiling_on_sc=False),
        out_type=(
            jax.ShapeDtypeStruct((_S,), jnp.float32),
            jax.ShapeDtypeStruct((_NBLK, 4, _BLK), jnp.float32),
            jax.ShapeDtypeStruct((_S,), jnp.float32),
            jax.ShapeDtypeStruct((_NBLK, _DP, _BLK), jnp.float32),
        ),
        scratch_types=[
            pltpu.VMEM((_BLK,), jnp.int32),
            pltpu.VMEM((_BLK,), jnp.int32),
            pltpu.VMEM((_BLK,), jnp.int32),
            pltpu.VMEM((_BLK,), jnp.int32),
            pltpu.VMEM((4, _BLK), jnp.int32),
            pltpu.VMEM((_DP, _BLK), jnp.int32),
            pltpu.VMEM((_BLK,), jnp.float32),
            pltpu.VMEM((4, _BLK), jnp.float32),
            pltpu.VMEM((_BLK,), jnp.float32),
            pltpu.VMEM((_DP, _BLK), jnp.float32),
            pltpu.SemaphoreType.DMA,
        ],
    )
    def gather_kernel(boxes_hbm, scores_hbm, joints_hbm,
                      selb_hbm, sell_hbm, selx_hbm,
                      out_b, out_boxes, out_sc, out_pose,
                      b_v, l_v, x_v, sidx_v, bidx_v, pidx_v,
                      bf_v, boxg_v, scg_v, poseg_v, sem):
        wid = lax.axis_index("s") * _NC + lax.axis_index("c")
        for t in range(_ROUNDS):
            blk = wid + _NW * t

            @pl.when(blk < _NBLK)
            def _round():
                base = blk * _BLK
                pltpu.sync_copy(selb_hbm.at[pl.ds(base, _BLK)], b_v)
                pltpu.sync_copy(sell_hbm.at[pl.ds(base, _BLK)], l_v)
                pltpu.sync_copy(selx_hbm.at[pl.ds(base, _BLK)], x_v)
                for i in range(_G):
                    sl = pl.ds(i * _L, _L)
                    b = b_v[sl]
                    x = x_v[sl]
                    flat = b * _N + x
                    bf_v[sl] = b.astype(jnp.float32)
                    sidx_v[sl] = flat + l_v[sl] * _N
                    fb = b * (4 * _N) + x
                    for c in range(4):
                        bidx_v[c, sl] = fb + c * _N
                    for c in range(_DP):
                        pidx_v[c, sl] = flat + c * (_B * _N)
                pend = [pltpu.async_copy(scores_hbm.at[sidx_v], scg_v, sem)]
                for c in range(4):
                    pend.append(pltpu.async_copy(
                        boxes_hbm.at[bidx_v.at[c]], boxg_v.at[c], sem))
                for c in range(_DP):
                    pend.append(pltpu.async_copy(
                        joints_hbm.at[pidx_v.at[c]], poseg_v.at[c], sem))
                for d in pend:
                    d.wait()
                pltpu.sync_copy(bf_v, out_b.at[pl.ds(base, _BLK)])
                pltpu.sync_copy(boxg_v, out_boxes.at[blk])
                pltpu.sync_copy(scg_v, out_sc.at[pl.ds(base, _BLK)])
                pltpu.sync_copy(poseg_v, out_pose.at[blk])

    return gather_kernel


_gather = _make_gather()


@jax.jit
def kernel(pred_boxes, pred_scores, pred_joints, selected_indexes):
    boxes_flat = jnp.transpose(pred_boxes, (0, 2, 1)).reshape(_B * 4 * _N)
    scores_flat = jnp.transpose(pred_scores, (0, 2, 1)).reshape(_B * _N)
    joints_flat = jnp.transpose(pred_joints, (2, 3, 0, 1)).reshape(
        _DP * _B * _N)
    bf, boxes_t, sc, pose_t = _gather(
        boxes_flat, scores_flat, joints_flat,
        selected_indexes[:, 0], selected_indexes[:, 1], selected_indexes[:, 2])
    boxes = boxes_t.transpose(0, 2, 1).reshape(_S, 4)
    pose = pose_t.transpose(0, 2, 1).reshape(_S, _DP)
    return jnp.concatenate([bf[:, None], boxes, sc[:, None], pose], axis=1)
